# Initial kernel scaffold; baseline (speedup 1.0000x reference)
import jax
import jax.numpy as jnp
from jax.experimental import pallas as pl


def kernel(A1_tensor, edge_index, edge_values, Lin1, Lin1_bias, n, W1, b1, W2, b2, W3, b3):
    raise NotImplementedError("write your pallas kernel here")


# speedup vs baseline: 5.9758x; 5.9758x over previous
import functools

import jax
import jax.numpy as jnp
from jax import lax
from jax.experimental import pallas as pl
from jax.experimental.pallas import tpu as pltpu
from jax.experimental.pallas import tpu_sc as plsc

N = 10000
E = 320000
D = 128

NC = 2
NS = 16
NW = NC * NS
EPT = E // NW
CH = 80
NCHUNK = EPT // CH
G = 25
NGRP = NCHUNK // G
NPAD = 10240
RPT = NPAD // NS
ZR = 40


def _spmm_sc(h, row3, col3, ev3):
    mesh = plsc.VectorSubcoreMesh(core_axis_name="c", subcore_axis_name="s")

    @functools.partial(
        pl.kernel,
        out_type=jax.ShapeDtypeStruct((NC, NPAD, D), jnp.float32),
        mesh=mesh,
        scratch_types=[
            pltpu.VMEM((G, CH), jnp.int32),
            pltpu.VMEM((G, CH), jnp.int32),
            pltpu.VMEM((G * CH,), jnp.float32),
            pltpu.VMEM((CH, D), jnp.float32),
            pltpu.VMEM((ZR, D), jnp.float32),
            pltpu.VMEM_SHARED((NPAD, D), jnp.float32),
            pltpu.SemaphoreType.DMA,
        ],
    )
    def k(h_hbm, row_hbm, col_hbm, ev_hbm, out_hbm,
          row_v, col_v, ev_v, rows_v, zbuf, acc, sem):
        sid = lax.axis_index("s")
        cid = lax.axis_index("c")
        wid = cid * NS + sid

        zero = jnp.zeros((16,), jnp.float32)

        def zb(i, _):
            zbuf[i // 8, pl.ds((i % 8) * 16, 16)] = zero
            return 0

        lax.fori_loop(0, ZR * 8, zb, 0)
        base = sid * RPT
        for t in range(RPT // ZR):
            pltpu.sync_copy(zbuf, acc.at[pl.ds(base + t * ZR, ZR)])
        plsc.subcore_barrier()

        def grp(gg, _):
            pltpu.sync_copy(row_hbm.at[wid, gg], row_v)
            pltpu.sync_copy(col_hbm.at[wid, gg], col_v)
            pltpu.sync_copy(ev_hbm.at[wid, gg], ev_v)

            def body(j, _):
                pltpu.async_copy(h_hbm.at[col_v.at[j]], rows_v, sem).wait()

                def ebody(g, _):
                    ebase = g * 16
                    evv16 = ev_v[pl.ds(j * CH + ebase, 16)]
                    for e in range(16):
                        evb = evv16.at[jnp.full((16,), e, jnp.int32)].get(
                            mode="promise_in_bounds")
                        r = ebase + e
                        for q in range(8):
                            rows_v[r, pl.ds(q * 16, 16)] = (
                                rows_v[r, pl.ds(q * 16, 16)] * evb)
                    return 0

                lax.fori_loop(0, CH // 16, ebody, 0)
                pltpu.sync_copy(rows_v, acc.at[row_v.at[j]], add=True)
                return 0

            lax.fori_loop(0, G, body, 0)
            return 0

        lax.fori_loop(0, NGRP, grp, 0)
        plsc.subcore_barrier()
        pltpu.sync_copy(acc.at[pl.ds(base, RPT)],
                        out_hbm.at[cid, pl.ds(base, RPT)])

    return k(h, row3, col3, ev3)


BM = 1000


def _mm2(x, Wa, ba, Wb, bb):
    def kfn(x_ref, wa_ref, ba_ref, wb_ref, bb_ref, a_ref, h_ref):
        xb = x_ref[...]
        a_ref[...] = jnp.dot(xb, wa_ref[...], preferred_element_type=jnp.float32) + ba_ref[...]
        h_ref[...] = jnp.dot(xb, wb_ref[...], preferred_element_type=jnp.float32) + bb_ref[...]

    return pl.pallas_call(
        kfn,
        grid=(N // BM,),
        in_specs=[
            pl.BlockSpec((BM, D), lambda i: (i, 0)),
            pl.BlockSpec((D, D), lambda i: (0, 0)),
            pl.BlockSpec((1, D), lambda i: (0, 0)),
            pl.BlockSpec((D, D), lambda i: (0, 0)),
            pl.BlockSpec((1, D), lambda i: (0, 0)),
        ],
        out_specs=[
            pl.BlockSpec((BM, D), lambda i: (i, 0)),
            pl.BlockSpec((BM, D), lambda i: (i, 0)),
        ],
        out_shape=[jax.ShapeDtypeStruct((N, D), jnp.float32)] * 2,
    )(x, Wa, ba.reshape(1, D), Wb, bb.reshape(1, D))


def _gate_mm(a0, nvec, s0, s1, W, b):
    def kfn(a0_ref, n_ref, s0_ref, s1_ref, w_ref, b_ref, h_ref):
        nb = n_ref[...]
        x1 = a0_ref[...] * nb + (s0_ref[...] + s1_ref[...]) * (1.0 - nb)
        h_ref[...] = jnp.dot(x1, w_ref[...], preferred_element_type=jnp.float32) + b_ref[...]

    return pl.pallas_call(
        kfn,
        grid=(N // BM,),
        in_specs=[
            pl.BlockSpec((BM, D), lambda i: (i, 0)),
            pl.BlockSpec((BM, 1), lambda i: (i, 0)),
            pl.BlockSpec((BM, D), lambda i: (i, 0)),
            pl.BlockSpec((BM, D), lambda i: (i, 0)),
            pl.BlockSpec((D, D), lambda i: (0, 0)),
            pl.BlockSpec((1, D), lambda i: (0, 0)),
        ],
        out_specs=pl.BlockSpec((BM, D), lambda i: (i, 0)),
        out_shape=jax.ShapeDtypeStruct((N, D), jnp.float32),
    )(a0, nvec, s0, s1, W, b.reshape(1, D))


def _add_mm(s0, s1, W, b):
    def kfn(s0_ref, s1_ref, w_ref, b_ref, h_ref):
        x = s0_ref[...] + s1_ref[...]
        h_ref[...] = jnp.dot(x, w_ref[...], preferred_element_type=jnp.float32) + b_ref[...]

    return pl.pallas_call(
        kfn,
        grid=(N // BM,),
        in_specs=[
            pl.BlockSpec((BM, D), lambda i: (i, 0)),
            pl.BlockSpec((BM, D), lambda i: (i, 0)),
            pl.BlockSpec((D, D), lambda i: (0, 0)),
            pl.BlockSpec((1, D), lambda i: (0, 0)),
        ],
        out_specs=pl.BlockSpec((BM, D), lambda i: (i, 0)),
        out_shape=jax.ShapeDtypeStruct((N, D), jnp.float32),
    )(s0, s1, W, b.reshape(1, D))


def _finalize(s0, s1, a0):
    def kfn(s0_ref, s1_ref, a0_ref, o_ref):
        o_ref[:, :D] = s0_ref[...] + s1_ref[...]
        o_ref[:, D:] = a0_ref[...]

    return pl.pallas_call(
        kfn,
        grid=(N // BM,),
        in_specs=[
            pl.BlockSpec((BM, D), lambda i: (i, 0)),
            pl.BlockSpec((BM, D), lambda i: (i, 0)),
            pl.BlockSpec((BM, D), lambda i: (i, 0)),
        ],
        out_specs=pl.BlockSpec((BM, 2 * D), lambda i: (i, 0)),
        out_shape=jax.ShapeDtypeStruct((N, 2 * D), jnp.float32),
    )(s0, s1, a0)


def kernel(A1_tensor, edge_index, edge_values, Lin1, Lin1_bias, n, W1, b1,
           W2, b2, W3, b3):
    x = A1_tensor[:, 1:]
    row3 = edge_index[0].reshape(NW, NGRP, G, CH)
    col3 = edge_index[1].reshape(NW, NGRP, G, CH)
    ev3 = edge_values.reshape(NW, NGRP, G * CH)

    a0, h1 = _mm2(x, Lin1, Lin1_bias, W1, b1)
    s1 = _spmm_sc(h1, row3, col3, ev3)
    h2 = _gate_mm(a0, n, s1[0, :N], s1[1, :N], W2, b2)
    s2 = _spmm_sc(h2, row3, col3, ev3)
    h3 = _add_mm(s2[0, :N], s2[1, :N], W3, b3)
    s3 = _spmm_sc(h3, row3, col3, ev3)
    return _finalize(s3[0, :N], s3[1, :N], a0)


# speedup vs baseline: 7.0902x; 1.1865x over previous
import functools

import jax
import jax.numpy as jnp
from jax import lax
from jax.experimental import pallas as pl
from jax.experimental.pallas import tpu as pltpu
from jax.experimental.pallas import tpu_sc as plsc

N = 10000
E = 320000
D = 128

NC = 2
NS = 16
NW = NC * NS
EPT = E // NW
CH = 100
NCHUNK = EPT // CH
G = 20
NGRP = NCHUNK // G
EVP = 112
NPAD = 10240
RPT = NPAD // NS
ZR = 16


def _spmm_sc(h, row3, col3, ev3):
    mesh = plsc.VectorSubcoreMesh(core_axis_name="c", subcore_axis_name="s")

    @functools.partial(
        pl.kernel,
        out_type=jax.ShapeDtypeStruct((NC, NPAD, D), jnp.float32),
        mesh=mesh,
        scratch_types=[
            pltpu.VMEM((G, CH), jnp.int32),
            pltpu.VMEM((G, CH), jnp.int32),
            pltpu.VMEM((G, EVP), jnp.float32),
            pltpu.VMEM((CH, D), jnp.float32),
            pltpu.VMEM((CH, D), jnp.float32),
            pltpu.VMEM((ZR, D), jnp.float32),
            pltpu.VMEM_SHARED((NPAD, D), jnp.float32),
            pltpu.SemaphoreType.DMA,
            pltpu.SemaphoreType.DMA,
            pltpu.SemaphoreType.DMA,
            pltpu.SemaphoreType.DMA,
        ],
    )
    def k(h_hbm, row_hbm, col_hbm, ev_hbm, out_hbm,
          row_v, col_v, ev_v, rows0, rows1, zbuf, acc,
          gsem0, gsem1, ssem0, ssem1):
        sid = lax.axis_index("s")
        cid = lax.axis_index("c")
        wid = cid * NS + sid

        zero = jnp.zeros((16,), jnp.float32)

        def zb(i, _):
            zbuf[i // 8, pl.ds((i % 8) * 16, 16)] = zero
            return 0

        lax.fori_loop(0, ZR * 8, zb, 0)
        base = sid * RPT
        for t in range(RPT // ZR):
            pltpu.async_copy(zbuf, acc.at[pl.ds(base + t * ZR, ZR)], gsem0)
        for t in range(RPT // ZR):
            pltpu.make_async_copy(zbuf, acc.at[pl.ds(base, ZR)], gsem0).wait()
        plsc.subcore_barrier()

        def scale(buf, j):
            for g in range(CH // 16 + 1):
                ne = 16 if (g + 1) * 16 <= CH else CH - g * 16
                if ne <= 0:
                    break
                evv16 = ev_v[j, pl.ds(g * 16, 16)]
                for e in range(ne):
                    evb = evv16.at[jnp.full((16,), e, jnp.int32)].get(
                        mode="promise_in_bounds")
                    r = g * 16 + e
                    for q in range(8):
                        buf[r, pl.ds(q * 16, 16)] = (
                            buf[r, pl.ds(q * 16, 16)] * evb)

        def grp(gg, _):
            pltpu.sync_copy(row_hbm.at[wid, gg], row_v)
            pltpu.sync_copy(col_hbm.at[wid, gg], col_v)
            pltpu.sync_copy(ev_hbm.at[wid, gg], ev_v)

            pltpu.async_copy(h_hbm.at[col_v.at[0]], rows0, gsem0)
            pltpu.async_copy(h_hbm.at[col_v.at[1]], rows1, gsem1)

            def pair(p, _):
                j0 = 2 * p
                j1 = 2 * p + 1
                pltpu.make_async_copy(h_hbm.at[col_v.at[j0]], rows0, gsem0).wait()
                scale(rows0, j0)
                pltpu.async_copy(rows0, acc.at[row_v.at[j0]], ssem0, add=True)
                pltpu.make_async_copy(h_hbm.at[col_v.at[j1]], rows1, gsem1).wait()
                scale(rows1, j1)
                pltpu.async_copy(rows1, acc.at[row_v.at[j1]], ssem1, add=True)

                @pl.when(p < G // 2 - 1)
                def _():
                    pltpu.make_async_copy(rows0, acc.at[row_v.at[j0]], ssem0).wait()
                    pltpu.async_copy(h_hbm.at[col_v.at[j0 + 2]], rows0, gsem0)
                    pltpu.make_async_copy(rows1, acc.at[row_v.at[j1]], ssem1).wait()
                    pltpu.async_copy(h_hbm.at[col_v.at[j1 + 2]], rows1, gsem1)

                return 0

            lax.fori_loop(0, G // 2, pair, 0)
            pltpu.make_async_copy(rows0, acc.at[row_v.at[G - 2]], ssem0).wait()
            pltpu.make_async_copy(rows1, acc.at[row_v.at[G - 1]], ssem1).wait()
            return 0

        lax.fori_loop(0, NGRP, grp, 0)
        plsc.subcore_barrier()
        pltpu.sync_copy(acc.at[pl.ds(base, RPT)],
                        out_hbm.at[cid, pl.ds(base, RPT)])

    return k(h, row3, col3, ev3)


BM = 1000


def _mm2(x, Wa, ba, Wb, bb):
    def kfn(x_ref, wa_ref, ba_ref, wb_ref, bb_ref, a_ref, h_ref):
        xb = x_ref[...]
        a_ref[...] = jnp.dot(xb, wa_ref[...], preferred_element_type=jnp.float32) + ba_ref[...]
        h_ref[...] = jnp.dot(xb, wb_ref[...], preferred_element_type=jnp.float32) + bb_ref[...]

    return pl.pallas_call(
        kfn,
        grid=(N // BM,),
        in_specs=[
            pl.BlockSpec((BM, D), lambda i: (i, 0)),
            pl.BlockSpec((D, D), lambda i: (0, 0)),
            pl.BlockSpec((1, D), lambda i: (0, 0)),
            pl.BlockSpec((D, D), lambda i: (0, 0)),
            pl.BlockSpec((1, D), lambda i: (0, 0)),
        ],
        out_specs=[
            pl.BlockSpec((BM, D), lambda i: (i, 0)),
            pl.BlockSpec((BM, D), lambda i: (i, 0)),
        ],
        out_shape=[jax.ShapeDtypeStruct((N, D), jnp.float32)] * 2,
    )(x, Wa, ba.reshape(1, D), Wb, bb.reshape(1, D))


def _gate_mm(a0, nvec, s0, s1, W, b):
    def kfn(a0_ref, n_ref, s0_ref, s1_ref, w_ref, b_ref, h_ref):
        nb = n_ref[...]
        x1 = a0_ref[...] * nb + (s0_ref[...] + s1_ref[...]) * (1.0 - nb)
        h_ref[...] = jnp.dot(x1, w_ref[...], preferred_element_type=jnp.float32) + b_ref[...]

    return pl.pallas_call(
        kfn,
        grid=(N // BM,),
        in_specs=[
            pl.BlockSpec((BM, D), lambda i: (i, 0)),
            pl.BlockSpec((BM, 1), lambda i: (i, 0)),
            pl.BlockSpec((BM, D), lambda i: (i, 0)),
            pl.BlockSpec((BM, D), lambda i: (i, 0)),
            pl.BlockSpec((D, D), lambda i: (0, 0)),
            pl.BlockSpec((1, D), lambda i: (0, 0)),
        ],
        out_specs=pl.BlockSpec((BM, D), lambda i: (i, 0)),
        out_shape=jax.ShapeDtypeStruct((N, D), jnp.float32),
    )(a0, nvec, s0, s1, W, b.reshape(1, D))


def _add_mm(s0, s1, W, b):
    def kfn(s0_ref, s1_ref, w_ref, b_ref, h_ref):
        x = s0_ref[...] + s1_ref[...]
        h_ref[...] = jnp.dot(x, w_ref[...], preferred_element_type=jnp.float32) + b_ref[...]

    return pl.pallas_call(
        kfn,
        grid=(N // BM,),
        in_specs=[
            pl.BlockSpec((BM, D), lambda i: (i, 0)),
            pl.BlockSpec((BM, D), lambda i: (i, 0)),
            pl.BlockSpec((D, D), lambda i: (0, 0)),
            pl.BlockSpec((1, D), lambda i: (0, 0)),
        ],
        out_specs=pl.BlockSpec((BM, D), lambda i: (i, 0)),
        out_shape=jax.ShapeDtypeStruct((N, D), jnp.float32),
    )(s0, s1, W, b.reshape(1, D))


def _finalize(s0, s1, a0):
    def kfn(s0_ref, s1_ref, a0_ref, o_ref):
        o_ref[:, :D] = s0_ref[...] + s1_ref[...]
        o_ref[:, D:] = a0_ref[...]

    return pl.pallas_call(
        kfn,
        grid=(N // BM,),
        in_specs=[
            pl.BlockSpec((BM, D), lambda i: (i, 0)),
            pl.BlockSpec((BM, D), lambda i: (i, 0)),
            pl.BlockSpec((BM, D), lambda i: (i, 0)),
        ],
        out_specs=pl.BlockSpec((BM, 2 * D), lambda i: (i, 0)),
        out_shape=jax.ShapeDtypeStruct((N, 2 * D), jnp.float32),
    )(s0, s1, a0)


def kernel(A1_tensor, edge_index, edge_values, Lin1, Lin1_bias, n, W1, b1,
           W2, b2, W3, b3):
    x = A1_tensor[:, 1:]
    row3 = edge_index[0].reshape(NW, NGRP, G, CH)
    col3 = edge_index[1].reshape(NW, NGRP, G, CH)
    ev3 = jnp.pad(edge_values.reshape(NW, NGRP, G, CH),
                  ((0, 0), (0, 0), (0, 0), (0, EVP - CH)))

    a0, h1 = _mm2(x, Lin1, Lin1_bias, W1, b1)
    s1 = _spmm_sc(h1, row3, col3, ev3)
    h2 = _gate_mm(a0, n, s1[0, :N], s1[1, :N], W2, b2)
    s2 = _spmm_sc(h2, row3, col3, ev3)
    h3 = _add_mm(s2[0, :N], s2[1, :N], W3, b3)
    s3 = _spmm_sc(h3, row3, col3, ev3)
    return _finalize(s3[0, :N], s3[1, :N], a0)


# speedup vs baseline: 7.5098x; 1.0592x over previous
import functools

import jax
import jax.numpy as jnp
from jax import lax
from jax.experimental import pallas as pl
from jax.experimental.pallas import tpu as pltpu
from jax.experimental.pallas import tpu_sc as plsc

N = 10000
E = 320000
D = 128

NC = 2
NS = 16
NW = NC * NS
EPT = E // NW
CH = 50
NCHUNK = EPT // CH
G = 40
NGRP = NCHUNK // G
NB = 4
EVP = 64
NPAD = 10240
RPT = NPAD // NS
ZR = 16


def _spmm_sc(h, row3, col3, ev3):
    mesh = plsc.VectorSubcoreMesh(core_axis_name="c", subcore_axis_name="s")

    @functools.partial(
        pl.kernel,
        out_type=jax.ShapeDtypeStruct((NC, NPAD, D), jnp.float32),
        mesh=mesh,
        scratch_types=[
            pltpu.VMEM((G, CH), jnp.int32),
            pltpu.VMEM((G, CH), jnp.int32),
            pltpu.VMEM((G, EVP), jnp.float32),
            pltpu.VMEM((CH, D), jnp.float32),
            pltpu.VMEM((CH, D), jnp.float32),
            pltpu.VMEM((CH, D), jnp.float32),
            pltpu.VMEM((CH, D), jnp.float32),
            pltpu.VMEM((ZR, D), jnp.float32),
            pltpu.VMEM_SHARED((NPAD, D), jnp.float32),
            pltpu.SemaphoreType.DMA,
            pltpu.SemaphoreType.DMA,
            pltpu.SemaphoreType.DMA,
            pltpu.SemaphoreType.DMA,
            pltpu.SemaphoreType.DMA,
            pltpu.SemaphoreType.DMA,
            pltpu.SemaphoreType.DMA,
            pltpu.SemaphoreType.DMA,
        ],
    )
    def k(h_hbm, row_hbm, col_hbm, ev_hbm, out_hbm,
          row_v, col_v, ev_v, b0, b1, b2, b3, zbuf, acc,
          g0, g1, g2, g3, s0, s1, s2, s3):
        sid = lax.axis_index("s")
        cid = lax.axis_index("c")
        wid = cid * NS + sid
        bufs = [b0, b1, b2, b3]
        gsem = [g0, g1, g2, g3]
        ssem = [s0, s1, s2, s3]

        zero = jnp.zeros((16,), jnp.float32)

        def zb(i, _):
            zbuf[i // 8, pl.ds((i % 8) * 16, 16)] = zero
            return 0

        lax.fori_loop(0, ZR * 8, zb, 0)
        base = sid * RPT
        for t in range(RPT // ZR):
            pltpu.async_copy(zbuf, acc.at[pl.ds(base + t * ZR, ZR)], g0)
        for t in range(RPT // ZR):
            pltpu.make_async_copy(zbuf, acc.at[pl.ds(base, ZR)], g0).wait()
        plsc.subcore_barrier()

        def scale(buf, j):
            for g in range(CH // 16 + 1):
                ne = 16 if (g + 1) * 16 <= CH else CH - g * 16
                if ne <= 0:
                    break
                evv16 = ev_v[j, pl.ds(g * 16, 16)]
                for e in range(ne):
                    evb = evv16.at[jnp.full((16,), e, jnp.int32)].get(
                        mode="promise_in_bounds")
                    r = g * 16 + e
                    for q in range(8):
                        buf[r, pl.ds(q * 16, 16)] = (
                            buf[r, pl.ds(q * 16, 16)] * evb)

        def grp(gg, _):
            pltpu.sync_copy(row_hbm.at[wid, gg], row_v)
            pltpu.sync_copy(col_hbm.at[wid, gg], col_v)
            pltpu.sync_copy(ev_hbm.at[wid, gg], ev_v)

            pltpu.async_copy(h_hbm.at[col_v.at[0]], b0, g0)
            pltpu.async_copy(h_hbm.at[col_v.at[1]], b1, g1)

            def step(p, _):
                for ln in range(NB):
                    j = NB * p + ln
                    buf = bufs[ln]
                    nxt = (ln + 2) % NB
                    pltpu.make_async_copy(h_hbm.at[col_v.at[j]], buf,
                                          gsem[ln]).wait()
                    scale(buf, j)
                    pltpu.async_copy(buf, acc.at[row_v.at[j]], ssem[ln],
                                     add=True)

                    def launch_guarded():
                        pltpu.make_async_copy(
                            bufs[nxt], acc.at[row_v.at[j]], ssem[nxt]).wait()
                        pltpu.async_copy(h_hbm.at[col_v.at[j + 2]],
                                         bufs[nxt], gsem[nxt])

                    if ln < 2:
                        @pl.when(p == 0)
                        def _():
                            pltpu.async_copy(h_hbm.at[col_v.at[j + 2]],
                                             bufs[nxt], gsem[nxt])

                        @pl.when(p > 0)
                        def _():
                            launch_guarded()
                    else:
                        @pl.when(p < G // NB - 1)
                        def _():
                            launch_guarded()
                return 0

            lax.fori_loop(0, G // NB, step, 0)
            for ln in range(NB):
                pltpu.make_async_copy(bufs[ln], acc.at[row_v.at[G - NB + ln]],
                                      ssem[ln]).wait()
            return 0

        lax.fori_loop(0, NGRP, grp, 0)
        plsc.subcore_barrier()
        pltpu.sync_copy(acc.at[pl.ds(base, RPT)],
                        out_hbm.at[cid, pl.ds(base, RPT)])

    return k(h, row3, col3, ev3)


BM = 1000


def _mm2(x, Wa, ba, Wb, bb):
    def kfn(x_ref, wa_ref, ba_ref, wb_ref, bb_ref, a_ref, h_ref):
        xb = x_ref[...]
        a_ref[...] = jnp.dot(xb, wa_ref[...], preferred_element_type=jnp.float32) + ba_ref[...]
        h_ref[...] = jnp.dot(xb, wb_ref[...], preferred_element_type=jnp.float32) + bb_ref[...]

    return pl.pallas_call(
        kfn,
        grid=(N // BM,),
        in_specs=[
            pl.BlockSpec((BM, D), lambda i: (i, 0)),
            pl.BlockSpec((D, D), lambda i: (0, 0)),
            pl.BlockSpec((1, D), lambda i: (0, 0)),
            pl.BlockSpec((D, D), lambda i: (0, 0)),
            pl.BlockSpec((1, D), lambda i: (0, 0)),
        ],
        out_specs=[
            pl.BlockSpec((BM, D), lambda i: (i, 0)),
            pl.BlockSpec((BM, D), lambda i: (i, 0)),
        ],
        out_shape=[jax.ShapeDtypeStruct((N, D), jnp.float32)] * 2,
    )(x, Wa, ba.reshape(1, D), Wb, bb.reshape(1, D))


def _gate_mm(a0, nvec, s0, s1, W, b):
    def kfn(a0_ref, n_ref, s0_ref, s1_ref, w_ref, b_ref, h_ref):
        nb = n_ref[...]
        x1 = a0_ref[...] * nb + (s0_ref[...] + s1_ref[...]) * (1.0 - nb)
        h_ref[...] = jnp.dot(x1, w_ref[...], preferred_element_type=jnp.float32) + b_ref[...]

    return pl.pallas_call(
        kfn,
        grid=(N // BM,),
        in_specs=[
            pl.BlockSpec((BM, D), lambda i: (i, 0)),
            pl.BlockSpec((BM, 1), lambda i: (i, 0)),
            pl.BlockSpec((BM, D), lambda i: (i, 0)),
            pl.BlockSpec((BM, D), lambda i: (i, 0)),
            pl.BlockSpec((D, D), lambda i: (0, 0)),
            pl.BlockSpec((1, D), lambda i: (0, 0)),
        ],
        out_specs=pl.BlockSpec((BM, D), lambda i: (i, 0)),
        out_shape=jax.ShapeDtypeStruct((N, D), jnp.float32),
    )(a0, nvec, s0, s1, W, b.reshape(1, D))


def _add_mm(s0, s1, W, b):
    def kfn(s0_ref, s1_ref, w_ref, b_ref, h_ref):
        x = s0_ref[...] + s1_ref[...]
        h_ref[...] = jnp.dot(x, w_ref[...], preferred_element_type=jnp.float32) + b_ref[...]

    return pl.pallas_call(
        kfn,
        grid=(N // BM,),
        in_specs=[
            pl.BlockSpec((BM, D), lambda i: (i, 0)),
            pl.BlockSpec((BM, D), lambda i: (i, 0)),
            pl.BlockSpec((D, D), lambda i: (0, 0)),
            pl.BlockSpec((1, D), lambda i: (0, 0)),
        ],
        out_specs=pl.BlockSpec((BM, D), lambda i: (i, 0)),
        out_shape=jax.ShapeDtypeStruct((N, D), jnp.float32),
    )(s0, s1, W, b.reshape(1, D))


def _finalize(s0, s1, a0):
    def kfn(s0_ref, s1_ref, a0_ref, o_ref):
        o_ref[:, :D] = s0_ref[...] + s1_ref[...]
        o_ref[:, D:] = a0_ref[...]

    return pl.pallas_call(
        kfn,
        grid=(N // BM,),
        in_specs=[
            pl.BlockSpec((BM, D), lambda i: (i, 0)),
            pl.BlockSpec((BM, D), lambda i: (i, 0)),
            pl.BlockSpec((BM, D), lambda i: (i, 0)),
        ],
        out_specs=pl.BlockSpec((BM, 2 * D), lambda i: (i, 0)),
        out_shape=jax.ShapeDtypeStruct((N, 2 * D), jnp.float32),
    )(s0, s1, a0)


def kernel(A1_tensor, edge_index, edge_values, Lin1, Lin1_bias, n, W1, b1,
           W2, b2, W3, b3):
    x = A1_tensor[:, 1:]
    row3 = edge_index[0].reshape(NW, NGRP, G, CH)
    col3 = edge_index[1].reshape(NW, NGRP, G, CH)
    ev3 = jnp.pad(edge_values.reshape(NW, NGRP, G, CH),
                  ((0, 0), (0, 0), (0, 0), (0, EVP - CH)))

    a0, h1 = _mm2(x, Lin1, Lin1_bias, W1, b1)
    s1 = _spmm_sc(h1, row3, col3, ev3)
    h2 = _gate_mm(a0, n, s1[0, :N], s1[1, :N], W2, b2)
    s2 = _spmm_sc(h2, row3, col3, ev3)
    h3 = _add_mm(s2[0, :N], s2[1, :N], W3, b3)
    s3 = _spmm_sc(h3, row3, col3, ev3)
    return _finalize(s3[0, :N], s3[1, :N], a0)
